# Initial kernel scaffold; baseline (speedup 1.0000x reference)
import jax
import jax.numpy as jnp
from jax.experimental import pallas as pl


def kernel(mem, idx, val, sample_idx):
    raise NotImplementedError("write your pallas kernel here")


# speedup vs baseline: 2.0304x; 2.0304x over previous
import jax
import jax.numpy as jnp
from jax import lax
from jax.experimental import pallas as pl
from jax.experimental.pallas import tpu as pltpu
from jax.experimental.pallas import tpu_sc as plsc

M = 1_000_000
D = 64
B = 16384
TBL = 1 << 20
NC, NS = 2, 16
NW = NC * NS
SB = B // NW
IB = B // NS
ROUNDS = 3
FILL = 8192
REG = TBL // NS
OUT_ROWS = B + SB


def _body(mem_hbm, idx_hbm, val_hbm, samp_hbm, out_hbm,
          tbl_sh, fill_v, idxs2, jv2, cur2, tgt2,
          samp2, p2, pc2, g2, rows_v, tmp_v):
    c = lax.axis_index("c")
    s = lax.axis_index("s")
    wid = s * NC + c
    ii16 = lax.iota(jnp.int32, 16)
    neg1 = jnp.full((16,), -1, jnp.int32)

    def _fill(i, _):
        fill_v[pl.ds(i * 16, 16)] = neg1
        return _
    lax.fori_loop(0, FILL // 16, _fill, 0)
    for b in range(REG // FILL):
        pltpu.sync_copy(fill_v, tbl_sh.at[pl.ds(s * REG + b * FILL, FILL)])

    for q in range(IB // 128):
        pltpu.sync_copy(idx_hbm.at[pl.ds(s * IB + q * 128, 128)], idxs2.at[q])
        for t in range(8):
            jv2[q, pl.ds(t * 16, 16)] = s * IB + q * 128 + t * 16 + ii16

    plsc.subcore_barrier()

    dump_cell = jnp.full((16,), M, jnp.int32) + wid
    for _ in range(ROUNDS):
        for q in range(IB // 128):
            pltpu.sync_copy(tbl_sh.at[idxs2.at[q]], cur2.at[q])
            for t in range(8):
                cu = cur2[q, pl.ds(t * 16, 16)]
                jj = jv2[q, pl.ds(t * 16, 16)]
                ix = idxs2[q, pl.ds(t * 16, 16)]
                tgt2[q, pl.ds(t * 16, 16)] = jnp.where(cu < jj, ix, dump_cell)
            pltpu.sync_copy(jv2.at[q], tbl_sh.at[tgt2.at[q]])
        plsc.subcore_barrier()

    for q in range(SB // 128):
        pltpu.sync_copy(samp_hbm.at[pl.ds(wid * SB + q * 128, 128)],
                        samp2.at[q])
        pltpu.sync_copy(tbl_sh.at[samp2.at[q]], p2.at[q])
        pltpu.sync_copy(mem_hbm.at[samp2.at[q]], rows_v)
        pltpu.sync_copy(rows_v, out_hbm.at[pl.ds(wid * SB + q * 128, 128)])
        for t in range(8):
            p = p2[q, pl.ds(t * 16, 16)]
            matched = p >= 0
            pc2[q, pl.ds(t * 16, 16)] = jnp.maximum(p, 0)
            here = wid * SB + q * 128 + t * 16 + ii16
            trash = B + q * 128 + t * 16 + ii16
            g2[q, pl.ds(t * 16, 16)] = jnp.where(matched, here, trash)
        pltpu.sync_copy(val_hbm.at[pc2.at[q]], tmp_v)
        pltpu.sync_copy(tmp_v, out_hbm.at[g2.at[q]])


def _build():
    mesh = plsc.VectorSubcoreMesh(core_axis_name="c", subcore_axis_name="s")
    return pl.kernel(
        _body,
        out_type=jax.ShapeDtypeStruct((OUT_ROWS, D), jnp.float32),
        mesh=mesh,
        compiler_params=pltpu.CompilerParams(use_tc_tiling_on_sc=False),
        scratch_types=[
            pltpu.VMEM_SHARED((TBL,), jnp.int32),
            pltpu.VMEM((FILL,), jnp.int32),
            pltpu.VMEM((IB // 128, 128), jnp.int32),
            pltpu.VMEM((IB // 128, 128), jnp.int32),
            pltpu.VMEM((IB // 128, 128), jnp.int32),
            pltpu.VMEM((IB // 128, 128), jnp.int32),
            pltpu.VMEM((SB // 128, 128), jnp.int32),
            pltpu.VMEM((SB // 128, 128), jnp.int32),
            pltpu.VMEM((SB // 128, 128), jnp.int32),
            pltpu.VMEM((SB // 128, 128), jnp.int32),
            pltpu.VMEM((128, D), jnp.float32),
            pltpu.VMEM((128, D), jnp.float32),
        ],
    )


_sc_kernel = _build()


def kernel(mem, idx, val, sample_idx):
    out = _sc_kernel(mem, idx, val, sample_idx)
    return out[:B]
